# Initial kernel scaffold; baseline (speedup 1.0000x reference)
import jax
import jax.numpy as jnp
from jax.experimental import pallas as pl


def kernel(boxes, scores):
    raise NotImplementedError("write your pallas kernel here")


# speedup vs baseline: 7.5854x; 7.5854x over previous
import jax
import jax.numpy as jnp
from jax.experimental import pallas as pl
from jax.experimental.pallas import tpu as pltpu

PRE_K = 1000
KPAD = 1024
OUT_K = 100
OUT_PAD = 128
IOU_THRESH = 0.5
NEG = -1e9
PAD_SCORE = -2e9


def _nms_body(bc_ref, br_ref, s_ref, out_ref, m_ref, sel_ref, sc_ref):
    x1c = bc_ref[:, 0:1]
    y1c = bc_ref[:, 1:2]
    x2c = bc_ref[:, 2:3]
    y2c = bc_ref[:, 3:4]
    x1r = br_ref[0:1, :]
    y1r = br_ref[1:2, :]
    x2r = br_ref[2:3, :]
    y2r = br_ref[3:4, :]

    area_c = (x2c - x1c) * (y2c - y1c)
    area_r = (x2r - x1r) * (y2r - y1r)
    iw = jnp.maximum(jnp.minimum(x2c, x2r) - jnp.maximum(x1c, x1r), 0.0)
    ih = jnp.maximum(jnp.minimum(y2c, y2r) - jnp.maximum(y1c, y1r), 0.0)
    inter = iw * ih
    union = area_c + area_r - inter
    iou = inter / jnp.maximum(union, 1e-9)

    row = jax.lax.broadcasted_iota(jnp.int32, (KPAD, KPAD), 0)
    col = jax.lax.broadcasted_iota(jnp.int32, (KPAD, KPAD), 1)
    m_ref[...] = jnp.where((iou > IOU_THRESH) & (col > row), 1.0, 0.0)

    colv = jax.lax.broadcasted_iota(jnp.int32, (1, KPAD), 1)

    def scan_step(i, supp):
        s_i = jnp.sum(jnp.where(colv == i, supp, 0.0))
        alive = 1.0 - s_i
        mrow = m_ref[pl.ds(i, 1), :]
        return jnp.maximum(supp, mrow * alive)

    supp = jax.lax.fori_loop(0, PRE_K, scan_step,
                             jnp.zeros((1, KPAD), jnp.float32))

    kept = jnp.where(supp > 0.0, NEG, s_ref[0:1, :])

    sel_ref[...] = jnp.zeros((OUT_PAD, KPAD), jnp.float32)

    def select_step(t, kept_t):
        m = jnp.max(kept_t)
        idx = jnp.min(jnp.where(kept_t == m, colv, KPAD))
        onehot = (colv == idx).astype(jnp.float32)
        sel_ref[pl.ds(t, 1), :] = onehot
        sc_ref[pl.ds(t, 1), 0:1] = jnp.full((1, 1), m, jnp.float32)
        return jnp.where(colv == idx, -1e30, kept_t)

    jax.lax.fori_loop(0, OUT_K, select_step, kept)

    res = jnp.dot(sel_ref[...], bc_ref[...],
                  preferred_element_type=jnp.float32)
    out_ref[...] = res
    out_ref[:, 4:5] = sc_ref[...]


def kernel(boxes, scores):
    top_scores, top_idx = jax.lax.top_k(scores, PRE_K)
    top_boxes = jnp.take(boxes, top_idx, axis=0)

    bc = jnp.zeros((KPAD, 8), jnp.float32).at[:PRE_K, :4].set(top_boxes)
    br = bc.T
    s = jnp.full((8, KPAD), PAD_SCORE, jnp.float32).at[0, :PRE_K].set(top_scores)

    out = pl.pallas_call(
        _nms_body,
        out_shape=jax.ShapeDtypeStruct((OUT_PAD, 8), jnp.float32),
        scratch_shapes=[
            pltpu.VMEM((KPAD, KPAD), jnp.float32),
            pltpu.VMEM((OUT_PAD, KPAD), jnp.float32),
            pltpu.VMEM((OUT_PAD, 1), jnp.float32),
        ],
    )(bc, br, s)
    return out[:OUT_K, :5]


# speedup vs baseline: 7.8144x; 1.0302x over previous
import jax
import jax.numpy as jnp
from jax.experimental import pallas as pl
from jax.experimental.pallas import tpu as pltpu

PRE_K = 1000
KPAD = 1024
BLK = 128
OUT_K = 100
OUT_PAD = 128
IOU_THRESH = 0.5
NEG = -1e9
PAD_SCORE = -2e9


def _nms_body(bc_ref, br_ref, s_ref, out_ref, m_ref, supp_ref):
    x1c = bc_ref[:, 0:1]
    y1c = bc_ref[:, 1:2]
    x2c = bc_ref[:, 2:3]
    y2c = bc_ref[:, 3:4]
    x1r = br_ref[0:1, :]
    y1r = br_ref[1:2, :]
    x2r = br_ref[2:3, :]
    y2r = br_ref[3:4, :]

    area_c = (x2c - x1c) * (y2c - y1c)
    area_r = (x2r - x1r) * (y2r - y1r)
    iw = jnp.maximum(jnp.minimum(x2c, x2r) - jnp.maximum(x1c, x1r), 0.0)
    ih = jnp.maximum(jnp.minimum(y2c, y2r) - jnp.maximum(y1c, y1r), 0.0)
    inter = iw * ih
    union = area_c + area_r - inter
    iou = inter / jnp.maximum(union, 1e-9)

    row = jax.lax.broadcasted_iota(jnp.int32, (KPAD, KPAD), 0)
    col = jax.lax.broadcasted_iota(jnp.int32, (KPAD, KPAD), 1)
    m_ref[...] = jnp.where((iou > IOU_THRESH) & (col > row), 1.0, 0.0)

    colvb = jax.lax.broadcasted_iota(jnp.int32, (1, BLK), 1)
    supp_ref[...] = jnp.zeros((1, KPAD), jnp.float32)

    for b in range(KPAD // BLK):
        base = b * BLK
        bsup0 = jnp.minimum(supp_ref[0:1, base:base + BLK], 1.0)

        def blk_step(k, bsup, base=base):
            s_k = jnp.sum(jnp.where(colvb == k, bsup, 0.0))
            alive = 1.0 - s_k
            mrow = m_ref[pl.ds(base + k, 1), :][:, base:base + BLK]
            return jnp.maximum(bsup, mrow * alive)

        bsup = jax.lax.fori_loop(0, BLK, blk_step, bsup0)

        alive_row = 1.0 - bsup
        mblk = m_ref[base:base + BLK, :]
        srow = jnp.dot(alive_row, mblk,
                       preferred_element_type=jnp.float32)
        new = jnp.maximum(supp_ref[0:1, :], srow)
        supp_ref[0:1, :] = new
        supp_ref[0:1, base:base + BLK] = jnp.maximum(
            new[:, base:base + BLK], bsup)

    kept = jnp.where(supp_ref[0:1, :] > 0.0, NEG, s_ref[0:1, :])
    kept2 = kept.reshape(8, 128)
    iota_abs = (jax.lax.broadcasted_iota(jnp.int32, (8, 128), 0) * 128
                + jax.lax.broadcasted_iota(jnp.int32, (8, 128), 1))

    def select_step(t, kept_t):
        m = jnp.max(kept_t)
        idx = jnp.min(jnp.where(kept_t == m, iota_abs, KPAD))
        out_ref[pl.ds(t, 1), 0:4] = bc_ref[pl.ds(idx, 1), 0:4]
        out_ref[pl.ds(t, 1), 4:5] = jnp.full((1, 1), m, jnp.float32)
        return jnp.where(iota_abs == idx, -1e30, kept_t)

    jax.lax.fori_loop(0, OUT_K, select_step, kept2)


def kernel(boxes, scores):
    top_scores, top_idx = jax.lax.top_k(scores, PRE_K)
    top_boxes = jnp.take(boxes, top_idx, axis=0)

    bc = jnp.zeros((KPAD, 8), jnp.float32).at[:PRE_K, :4].set(top_boxes)
    br = bc.T
    s = jnp.full((8, KPAD), PAD_SCORE, jnp.float32).at[0, :PRE_K].set(top_scores)

    out = pl.pallas_call(
        _nms_body,
        out_shape=jax.ShapeDtypeStruct((OUT_PAD, 8), jnp.float32),
        scratch_shapes=[
            pltpu.VMEM((KPAD, KPAD), jnp.float32),
            pltpu.VMEM((1, KPAD), jnp.float32),
        ],
    )(bc, br, s)
    return out[:OUT_K, :5]


# speedup vs baseline: 21.9728x; 2.8118x over previous
import jax
import jax.numpy as jnp
from jax.experimental import pallas as pl
from jax.experimental.pallas import tpu as pltpu

PRE_K = 1000
KPAD = 1024
BLK = 128
OUT_K = 100
OUT_PAD = 128
IOU_THRESH = 0.5
NEG = -1e9
PAD_SCORE = -2e9


def _nms_body(bc_ref, br_ref, s_ref, out_ref, m_ref):
    x1c = bc_ref[:, 0:1]
    y1c = bc_ref[:, 1:2]
    x2c = bc_ref[:, 2:3]
    y2c = bc_ref[:, 3:4]
    x1r = br_ref[0:1, :]
    y1r = br_ref[1:2, :]
    x2r = br_ref[2:3, :]
    y2r = br_ref[3:4, :]

    area_c = (x2c - x1c) * (y2c - y1c)
    area_r = (x2r - x1r) * (y2r - y1r)
    iw = jnp.maximum(jnp.minimum(x2c, x2r) - jnp.maximum(x1c, x1r), 0.0)
    ih = jnp.maximum(jnp.minimum(y2c, y2r) - jnp.maximum(y1c, y1r), 0.0)
    inter = iw * ih
    union = area_c + area_r - inter
    iou = inter / jnp.maximum(union, 1e-9)

    row = jax.lax.broadcasted_iota(jnp.int32, (KPAD, KPAD), 0)
    col = jax.lax.broadcasted_iota(jnp.int32, (KPAD, KPAD), 1)
    m_ref[...] = jnp.where((iou > IOU_THRESH) & (col > row), 1.0, 0.0)

    def fix_cond(state):
        return state[1]

    def fix_body(state):
        a, _ = state
        s = jnp.dot(a, m_ref[...],
                    preferred_element_type=jnp.float32)
        a_new = jnp.where(s > 0.0, 0.0, 1.0)
        return a_new, jnp.any(a_new != a)

    alive, _ = jax.lax.while_loop(
        fix_cond, fix_body,
        (jnp.ones((1, KPAD), jnp.float32), jnp.bool_(True)))

    kept = jnp.where(alive > 0.0, s_ref[0:1, :], NEG)
    kept2 = kept.reshape(8, 128)
    iota_abs = (jax.lax.broadcasted_iota(jnp.int32, (8, 128), 0) * 128
                + jax.lax.broadcasted_iota(jnp.int32, (8, 128), 1))

    def select_step(t, kept_t):
        m = jnp.max(kept_t)
        idx = jnp.min(jnp.where(kept_t == m, iota_abs, KPAD))
        out_ref[pl.ds(t, 1), 0:4] = bc_ref[pl.ds(idx, 1), 0:4]
        out_ref[pl.ds(t, 1), 4:5] = jnp.full((1, 1), m, jnp.float32)
        return jnp.where(iota_abs == idx, -1e30, kept_t)

    jax.lax.fori_loop(0, OUT_K, select_step, kept2)


def kernel(boxes, scores):
    top_scores, top_idx = jax.lax.top_k(scores, PRE_K)
    top_boxes = jnp.take(boxes, top_idx, axis=0)

    bc = jnp.zeros((KPAD, 8), jnp.float32).at[:PRE_K, :4].set(top_boxes)
    br = bc.T
    s = jnp.full((8, KPAD), PAD_SCORE, jnp.float32).at[0, :PRE_K].set(top_scores)

    out = pl.pallas_call(
        _nms_body,
        out_shape=jax.ShapeDtypeStruct((OUT_PAD, 8), jnp.float32),
        scratch_shapes=[
            pltpu.VMEM((KPAD, KPAD), jnp.float32),
        ],
    )(bc, br, s)
    return out[:OUT_K, :5]


# speedup vs baseline: 21.9856x; 1.0006x over previous
import jax
import jax.numpy as jnp
from jax.experimental import pallas as pl
from jax.experimental.pallas import tpu as pltpu

PRE_K = 1000
KPAD = 1024
BLK = 128
OUT_K = 100
OUT_PAD = 128
IOU_THRESH = 0.5
NEG = -1e9
PAD_SCORE = -2e9


def _nms_body(bc_ref, br_ref, s_ref, out_ref, m_ref):
    x1c = bc_ref[:, 0:1]
    y1c = bc_ref[:, 1:2]
    x2c = bc_ref[:, 2:3]
    y2c = bc_ref[:, 3:4]
    x1r = br_ref[0:1, :]
    y1r = br_ref[1:2, :]
    x2r = br_ref[2:3, :]
    y2r = br_ref[3:4, :]

    area_c = (x2c - x1c) * (y2c - y1c)
    area_r = (x2r - x1r) * (y2r - y1r)
    iw = jnp.maximum(jnp.minimum(x2c, x2r) - jnp.maximum(x1c, x1r), 0.0)
    ih = jnp.maximum(jnp.minimum(y2c, y2r) - jnp.maximum(y1c, y1r), 0.0)
    inter = iw * ih
    union = area_c + area_r - inter
    iou = inter / jnp.maximum(union, 1e-9)

    row = jax.lax.broadcasted_iota(jnp.int32, (KPAD, KPAD), 0)
    col = jax.lax.broadcasted_iota(jnp.int32, (KPAD, KPAD), 1)
    m_ref[...] = jnp.where((iou > IOU_THRESH) & (col > row),
                           1.0, 0.0).astype(jnp.bfloat16)

    def fix_cond(state):
        return state[1]

    def fix_body(state):
        a, _ = state
        s = jnp.dot(a.astype(jnp.bfloat16), m_ref[...],
                    preferred_element_type=jnp.float32)
        a_new = jnp.where(s > 0.0, 0.0, 1.0)
        return a_new, jnp.any(a_new != a)

    alive, _ = jax.lax.while_loop(
        fix_cond, fix_body,
        (jnp.ones((1, KPAD), jnp.float32), jnp.bool_(True)))

    kept = jnp.where(alive > 0.0, s_ref[0:1, :], NEG)
    kept2 = kept.reshape(8, 128)
    iota_abs = (jax.lax.broadcasted_iota(jnp.int32, (8, 128), 0) * 128
                + jax.lax.broadcasted_iota(jnp.int32, (8, 128), 1))

    def select_step(t, kept_t):
        m = jnp.max(kept_t)
        idx = jnp.min(jnp.where(kept_t == m, iota_abs, KPAD))
        out_ref[pl.ds(t, 1), 0:4] = bc_ref[pl.ds(idx, 1), 0:4]
        out_ref[pl.ds(t, 1), 4:5] = jnp.full((1, 1), m, jnp.float32)
        return jnp.where(iota_abs == idx, -1e30, kept_t)

    jax.lax.fori_loop(0, OUT_K, select_step, kept2)


def kernel(boxes, scores):
    top_scores, top_idx = jax.lax.top_k(scores, PRE_K)
    top_boxes = jnp.take(boxes, top_idx, axis=0)

    bc = jnp.zeros((KPAD, 8), jnp.float32).at[:PRE_K, :4].set(top_boxes)
    br = bc.T
    s = jnp.full((8, KPAD), PAD_SCORE, jnp.float32).at[0, :PRE_K].set(top_scores)

    out = pl.pallas_call(
        _nms_body,
        out_shape=jax.ShapeDtypeStruct((OUT_PAD, 8), jnp.float32),
        scratch_shapes=[
            pltpu.VMEM((KPAD, KPAD), jnp.bfloat16),
        ],
    )(bc, br, s)
    return out[:OUT_K, :5]


# speedup vs baseline: 21.9989x; 1.0006x over previous
import jax
import jax.numpy as jnp
from jax.experimental import pallas as pl
from jax.experimental.pallas import tpu as pltpu

PRE_K = 1000
KPAD = 1024
OUT_K = 100
OUT_PAD = 128
IOU_THRESH = 0.5
NEG = -1e9
PAD_SCORE = -2e9


def _nms_body(bc_ref, br_ref, s_ref, out_ref, m_ref):
    x1c = bc_ref[:, 0:1]
    y1c = bc_ref[:, 1:2]
    x2c = bc_ref[:, 2:3]
    y2c = bc_ref[:, 3:4]
    x1r = br_ref[0:1, :]
    y1r = br_ref[1:2, :]
    x2r = br_ref[2:3, :]
    y2r = br_ref[3:4, :]

    area_c = (x2c - x1c) * (y2c - y1c)
    area_r = (x2r - x1r) * (y2r - y1r)
    iw = jnp.maximum(jnp.minimum(x2c, x2r) - jnp.maximum(x1c, x1r), 0.0)
    ih = jnp.maximum(jnp.minimum(y2c, y2r) - jnp.maximum(y1c, y1r), 0.0)
    inter = iw * ih
    union = area_c + area_r - inter
    iou = inter / jnp.maximum(union, 1e-9)

    row = jax.lax.broadcasted_iota(jnp.int32, (KPAD, KPAD), 0)
    col = jax.lax.broadcasted_iota(jnp.int32, (KPAD, KPAD), 1)
    m_ref[...] = jnp.where((iou > IOU_THRESH) & (col > row),
                           1.0, 0.0).astype(jnp.bfloat16)

    def fix_cond(state):
        return state[1]

    def fix_body(state):
        a, _ = state
        s = jnp.dot(a.astype(jnp.bfloat16), m_ref[...],
                    preferred_element_type=jnp.float32)
        a_new = jnp.where(s > 0.0, 0.0, 1.0)
        return a_new, jnp.any(a_new != a)

    alive, _ = jax.lax.while_loop(
        fix_cond, fix_body,
        (jnp.ones((1, KPAD), jnp.float32), jnp.bool_(True)))

    kept = jnp.where(alive > 0.0, s_ref[0:1, :], NEG)
    kept2 = kept.reshape(8, 128)
    iota_abs = (jax.lax.broadcasted_iota(jnp.int32, (8, 128), 0) * 128
                + jax.lax.broadcasted_iota(jnp.int32, (8, 128), 1))

    def select_step(t, kept_t):
        m = jnp.max(kept_t)
        idx = jnp.min(jnp.where(kept_t == m, iota_abs, KPAD))
        out_ref[pl.ds(t, 1), 0:4] = bc_ref[pl.ds(idx, 1), 0:4]
        out_ref[pl.ds(t, 1), 4:5] = jnp.full((1, 1), m, jnp.float32)
        return jnp.where(iota_abs == idx, -1e30, kept_t)

    jax.lax.fori_loop(0, OUT_K, select_step, kept2)


def kernel(boxes, scores):
    top_scores, top_idx = jax.lax.top_k(scores, PRE_K)
    top_boxes = jnp.take(boxes, top_idx, axis=0)

    bc = jnp.zeros((KPAD, 8), jnp.float32).at[:PRE_K, :4].set(top_boxes)
    br = bc.T
    s = jnp.full((8, KPAD), PAD_SCORE, jnp.float32).at[0, :PRE_K].set(top_scores)

    out = pl.pallas_call(
        _nms_body,
        out_shape=jax.ShapeDtypeStruct((OUT_PAD, 8), jnp.float32),
        scratch_shapes=[
            pltpu.VMEM((KPAD, KPAD), jnp.bfloat16),
        ],
    )(bc, br, s)
    return out[:OUT_K, :5]
